# baseline (speedup 1.0000x reference)
import jax
import jax.numpy as jnp
from jax.experimental import pallas as pl
from jax.experimental.pallas import tpu as pltpu

B, T, C, L = 32, 500, 256, 50
S = 2 * L + 1
SP = 128
T_BLK = 50
NT = T // T_BLK
NORM = 5
DEN_SCALE = 1.0
NEG = -1e30


def _prep(x_ref, oh_ref, probs_ref, pemit_ref):
    x = x_ref[...]
    m = jnp.max(x, axis=2, keepdims=True)
    e = jnp.exp(x - m)
    sm = jnp.sum(e, axis=2, keepdims=True)
    lp = x - m - jnp.log(sm)
    probs_ref[...] = (e / sm).astype(jnp.bfloat16)
    lp16 = lp.astype(jnp.bfloat16)
    for b in range(B):
        pemit_ref[:, b, :] = jnp.dot(lp16[:, b, :], oh_ref[b],
                                     preferred_element_type=jnp.float32)


def _scan(i, probs_ref, pemit_ref, p_ref, skip,
          z_ref, lz_ref, na_ref):
    z = z_ref[...]
    lz = lz_ref[...]
    na = na_ref[...]
    for t in range(T_BLK):
        pt = probs_ref[t]
        et = pemit_ref[t]
        u = jnp.dot(z, p_ref[...],
                    preferred_element_type=jnp.float32).astype(jnp.bfloat16) * pt
        if t == 0:
            u = jnp.where(i == 1, pt * (1.0 / C), u)
        if t % NORM == NORM - 1:
            s = jnp.sum(u.astype(jnp.float32), axis=1, keepdims=True)
            z = u * (1.0 / s).astype(jnp.bfloat16)
            lz = lz + jnp.log(s)
        else:
            z = u
        a1 = jnp.concatenate([jnp.full((B, 1), NEG, jnp.float32), na[:, :-1]], axis=1)
        a2 = jnp.where(skip,
                       jnp.concatenate([jnp.full((B, 2), NEG, jnp.float32), na[:, :-2]], axis=1),
                       jnp.float32(NEG))
        mx = jnp.maximum(jnp.maximum(na, a1), a2)
        mxe = mx + et
        na = mxe + jnp.log(jnp.exp(na - mx) + jnp.exp(a1 - mx) + jnp.exp(a2 - mx))
    z_ref[...] = z
    lz_ref[...] = lz
    na_ref[...] = na


def _fwd_kernel(x_ref, ext_ref, skip_ref, dent_ref, frames_ref,
                out_s_ref, out_tf_ref, out_af_ref,
                p_ref, oh_ref, probs_a, pemit_a, probs_b, pemit_b,
                z_ref, lz_ref, na_ref):
    i = pl.program_id(0)

    @pl.when(i == 0)
    def _init():
        dt = dent_ref[...]
        m = jnp.max(dt, axis=1, keepdims=True)
        e = jnp.exp(dt - m)
        p_ref[...] = (e / jnp.sum(e, axis=1, keepdims=True)).astype(jnp.bfloat16)
        ids = jax.lax.broadcasted_iota(jnp.int32, (B, C, SP), 1)
        oh_ref[...] = (ids == ext_ref[...][:, None, :]).astype(jnp.bfloat16)
        z_ref[...] = jnp.zeros((B, C), jnp.bfloat16)
        lz_ref[...] = jnp.zeros((B, 1), jnp.float32)
        lane = jax.lax.broadcasted_iota(jnp.int32, (B, SP), 1)
        na_ref[...] = jnp.where(lane == 0, 0.0, NEG)

    @pl.when((i < NT) & (i % 2 == 0))
    def _prep_a():
        _prep(x_ref, oh_ref, probs_a, pemit_a)

    @pl.when((i < NT) & (i % 2 == 1))
    def _prep_b():
        _prep(x_ref, oh_ref, probs_b, pemit_b)

    skip = skip_ref[...] > 0

    @pl.when((i > 0) & (i % 2 == 1))
    def _scan_a():
        _scan(i, probs_a, pemit_a, p_ref, skip, z_ref, lz_ref, na_ref)

    @pl.when((i > 0) & (i % 2 == 0))
    def _scan_b():
        _scan(i, probs_b, pemit_b, p_ref, skip, z_ref, lz_ref, na_ref)

    @pl.when(i == NT)
    def _finish():
        na = na_ref[...]
        lz = lz_ref[...]
        aL, aK = na[:, S - 1:S], na[:, S - 2:S - 1]
        m2 = jnp.maximum(aL, aK)
        num_tot = m2 + jnp.log(jnp.exp(aL - m2) + jnp.exp(aK - m2))
        ts = num_tot - DEN_SCALE * lz
        fr = frames_ref[...][:, 0:1]
        mask = jnp.isfinite(ts) & (ts > -1e20)
        out_s_ref[...] = jnp.sum(jnp.where(mask, ts, 0.0)).reshape(1, 1)
        out_tf_ref[...] = jnp.sum(jnp.where(mask, fr, 0.0)).reshape(1, 1)
        out_af_ref[...] = jnp.sum(fr).reshape(1, 1)


@jax.jit
def kernel(nnet_output, labels, supervision_segments, den_trans):
    ext = jnp.full((B, SP), -1, jnp.int32)
    ext = ext.at[:, 0:S:2].set(0)
    ext = ext.at[:, 1:S:2].set(labels)
    ext_prev2 = jnp.concatenate([jnp.full((B, 2), -1, jnp.int32), ext[:, :-2]], axis=1)
    skip = ((ext > 0) & (ext != ext_prev2)).astype(jnp.float32)
    frames = supervision_segments[:, 2].astype(jnp.float32)
    frames_b = jnp.broadcast_to(frames[:, None], (B, SP))

    xt = jnp.transpose(nnet_output, (1, 0, 2))

    grid = (NT + 1,)
    out_s, out_tf, out_af = pl.pallas_call(
        _fwd_kernel,
        grid=grid,
        in_specs=[
            pl.BlockSpec((T_BLK, B, C), lambda i: (jnp.minimum(i, NT - 1), 0, 0)),
            pl.BlockSpec((B, SP), lambda i: (0, 0)),
            pl.BlockSpec((B, SP), lambda i: (0, 0)),
            pl.BlockSpec((C, C), lambda i: (0, 0)),
            pl.BlockSpec((B, SP), lambda i: (0, 0)),
        ],
        out_specs=[
            pl.BlockSpec((1, 1), lambda i: (0, 0)),
            pl.BlockSpec((1, 1), lambda i: (0, 0)),
            pl.BlockSpec((1, 1), lambda i: (0, 0)),
        ],
        out_shape=[
            jax.ShapeDtypeStruct((1, 1), jnp.float32),
            jax.ShapeDtypeStruct((1, 1), jnp.float32),
            jax.ShapeDtypeStruct((1, 1), jnp.float32),
        ],
        scratch_shapes=[
            pltpu.VMEM((C, C), jnp.bfloat16),
            pltpu.VMEM((B, C, SP), jnp.bfloat16),
            pltpu.VMEM((T_BLK, B, C), jnp.bfloat16),
            pltpu.VMEM((T_BLK, B, SP), jnp.float32),
            pltpu.VMEM((T_BLK, B, C), jnp.bfloat16),
            pltpu.VMEM((T_BLK, B, SP), jnp.float32),
            pltpu.VMEM((B, C), jnp.bfloat16),
            pltpu.VMEM((B, 1), jnp.float32),
            pltpu.VMEM((B, SP), jnp.float32),
        ],
        compiler_params=pltpu.CompilerParams(
            dimension_semantics=("arbitrary",),
        ),
    )(xt, ext, skip, den_trans, frames_b)

    return out_s[0, 0], out_tf[0, 0], out_af[0, 0]
